# baseline (speedup 1.0000x reference)
import functools

import jax
import jax.numpy as jnp
from jax import lax
from jax.experimental import pallas as pl
from jax.experimental.pallas import tpu as pltpu
from jax.experimental.pallas import tpu_sc as plsc

N = 10000
N_PAD = 10240
E = 320000
D_IN = 128
HID = 60
OUT = 64
EPS = 1e-5

NC = 2
NS = 16
NW = NC * NS
CH = 128
NCH = 80
IB = 16
NB = NCH // IB
E_PAD = NW * CH * NCH
STRIPE = N_PAD // NS

TILE_R = 1024
NT = N_PAD // TILE_R


def _sc_mesh():
    return plsc.VectorSubcoreMesh(core_axis_name="c", subcore_axis_name="s")


def _fill(rows_ref, value, width):
    vec = jnp.full((16,), value, jnp.float32)

    def body(i, _):
        for k in range(width // 16):
            rows_ref[i, pl.ds(k * 16, 16)] = vec
        return 0

    lax.fori_loop(0, CH, body, 0)


def _propagate(hs, srcr, dstr):
    W = hs.shape[1]

    @functools.partial(
        pl.kernel,
        out_type=jax.ShapeDtypeStruct((NC, N_PAD, W), jnp.float32),
        mesh=_sc_mesh(),
        scratch_types=[
            pltpu.VMEM((2, IB, CH), jnp.int32),
            pltpu.VMEM((2, IB, CH), jnp.int32),
            pltpu.VMEM((CH, W), jnp.float32),
            pltpu.VMEM((CH, W), jnp.float32),
            pltpu.VMEM_SHARED((N_PAD, W), jnp.float32),
            pltpu.SemaphoreType.DMA,
            pltpu.SemaphoreType.DMA,
            pltpu.SemaphoreType.DMA,
            pltpu.SemaphoreType.DMA,
        ],
    )
    def k(hs_ref, srcr_ref, dstr_ref, out_ref, sidxb, didxb, rows0, rows1,
          acc, gsem0, gsem1, isem0, isem1):
        c = lax.axis_index("c")
        s = lax.axis_index("s")
        wid = c * NS + s
        rows = (rows0, rows1)
        gsem = (gsem0, gsem1)
        isem = (isem0, isem1)

        def idx_fetch(kb, slot):
            base = kb * IB
            pltpu.async_copy(srcr_ref.at[wid, pl.ds(base, IB)],
                             sidxb.at[slot], isem[slot])
            pltpu.async_copy(dstr_ref.at[wid, pl.ds(base, IB)],
                             didxb.at[slot], isem[slot])

        def idx_wait(slot):
            for _ in range(2):
                pltpu.make_async_copy(srcr_ref.at[wid, pl.ds(0, IB)],
                                      sidxb.at[slot], isem[slot]).wait()

        def gath(slot, l, b):
            pltpu.async_copy(hs_ref.at[sidxb.at[slot, l]], rows[b], gsem[b])

        def wait_g(b):
            pltpu.make_async_copy(hs_ref.at[sidxb.at[0, 0]], rows[b],
                                  gsem[b]).wait()

        def scat(slot, l, b):
            pltpu.sync_copy(rows[b], acc.at[didxb.at[slot, l]], add=True)

        idx_fetch(0, 0)
        _fill(rows0, 0.0, W)
        for r in range(STRIPE // CH):
            pltpu.sync_copy(rows0, acc.at[pl.ds(s * STRIPE + r * CH, CH)])
        plsc.subcore_barrier()

        for kb in range(NB):
            slot = kb % 2
            idx_wait(slot)
            if kb + 1 < NB:
                idx_fetch(kb + 1, 1 - slot)
            gath(slot, 0, 0)
            gath(slot, 1, 1)

            def pair(a, _):
                wait_g(0)
                scat(slot, 2 * a, 0)
                gath(slot, 2 * a + 2, 0)
                wait_g(1)
                scat(slot, 2 * a + 1, 1)
                gath(slot, 2 * a + 3, 1)
                return 0

            lax.fori_loop(0, IB // 2 - 1, pair, 0,
                          unroll=False)
            wait_g(0)
            scat(slot, IB - 2, 0)
            wait_g(1)
            scat(slot, IB - 1, 1)

        plsc.subcore_barrier()
        pltpu.sync_copy(acc.at[pl.ds(s * STRIPE, STRIPE)],
                        out_ref.at[c, pl.ds(s * STRIPE, STRIPE)])

    return k(hs, srcr, dstr)


_TC_PARAMS = pltpu.CompilerParams(dimension_semantics=("arbitrary",))


def _dinv_tile(deg_ref):
    d = deg_ref[0, :, 0:1] + deg_ref[1, :, 0:1]
    return lax.rsqrt(jnp.maximum(d, 1.0))


def _stats_update(sacc, q, i):
    @pl.when(i == 0)
    def _():
        sacc[...] = jnp.zeros_like(sacc)

    sacc[0:1, :] += jnp.sum(q, axis=0, keepdims=True)
    sacc[1:2, :] += jnp.sum(q * q, axis=0, keepdims=True)


def _tc_first(x, degp, wc):
    def body(x_ref, deg_ref, w_ref, q0_ref, hs_ref, st_ref, sacc):
        i = pl.program_id(0)
        p = jnp.dot(x_ref[...], w_ref[...], preferred_element_type=jnp.float32)
        q0 = p[:, 0:64]
        q0_ref[...] = q0
        dinv = _dinv_tile(deg_ref)
        hs_ref[...] = p[:, 64:192] * dinv
        _stats_update(sacc, q0, i)

        @pl.when(i == NT - 1)
        def _():
            st_ref[...] = sacc[...]

    return pl.pallas_call(
        body,
        grid=(NT,),
        in_specs=[
            pl.BlockSpec((TILE_R, D_IN), lambda i: (i, 0)),
            pl.BlockSpec((2, TILE_R, 128), lambda i: (0, i, 0)),
            pl.BlockSpec((D_IN, 192), lambda i: (0, 0)),
        ],
        out_specs=[
            pl.BlockSpec((TILE_R, 64), lambda i: (i, 0)),
            pl.BlockSpec((TILE_R, 128), lambda i: (i, 0)),
            pl.BlockSpec((8, 64), lambda i: (0, 0)),
        ],
        out_shape=[
            jax.ShapeDtypeStruct((N_PAD, 64), jnp.float32),
            jax.ShapeDtypeStruct((N_PAD, 128), jnp.float32),
            jax.ShapeDtypeStruct((8, 64), jnp.float32),
        ],
        scratch_shapes=[pltpu.VMEM((8, 64), jnp.float32)],
        compiler_params=_TC_PARAMS,
    )(x, degp, wc)


def _bn_tanh(parts, stats, g_ref, b_ref):
    h = jnp.concatenate(parts, axis=1)
    sm = jnp.concatenate([s[0:1, :] for s in stats], axis=1)
    sq = jnp.concatenate([s[1:2, :] for s in stats], axis=1)
    m = sm * (1.0 / N)
    v = sq * (1.0 / N) - m * m
    return jnp.tanh((h - m) * lax.rsqrt(v + EPS) * g_ref[...] + b_ref[...])


def _tc_layer(p0, p1, p2, s0, s1, s2, gp, bp, degp, wc):
    def body(p0_ref, p1_ref, p2_ref, s0_ref, s1_ref, s2_ref, g_ref, b_ref,
             deg_ref, w_ref, q0_ref, hs_ref, st_ref, sacc):
        i = pl.program_id(0)
        hn = _bn_tanh((p0_ref[...], p1_ref[...], p2_ref[...]),
                      (s0_ref, s1_ref, s2_ref), g_ref, b_ref)
        rows = i * TILE_R + lax.broadcasted_iota(jnp.int32, (TILE_R, 1), 0)
        hn = jnp.where(rows < N, hn, 0.0)
        p = jnp.dot(hn, w_ref[...], preferred_element_type=jnp.float32)
        q0 = p[:, 0:64]
        q0_ref[...] = q0
        dinv = _dinv_tile(deg_ref)
        hs_ref[...] = p[:, 64:192] * dinv
        _stats_update(sacc, q0, i)

        @pl.when(i == NT - 1)
        def _():
            st_ref[...] = sacc[...]

    part = pl.BlockSpec((TILE_R, 64), lambda i: (i, 0))
    st_in = pl.BlockSpec((8, 64), lambda i: (0, 0))
    vec = pl.BlockSpec((1, 192), lambda i: (0, 0))
    return pl.pallas_call(
        body,
        grid=(NT,),
        in_specs=[part, part, part, st_in, st_in, st_in, vec, vec,
                  pl.BlockSpec((2, TILE_R, 128), lambda i: (0, i, 0)),
                  pl.BlockSpec((192, 192), lambda i: (0, 0))],
        out_specs=[
            pl.BlockSpec((TILE_R, 64), lambda i: (i, 0)),
            pl.BlockSpec((TILE_R, 128), lambda i: (i, 0)),
            pl.BlockSpec((8, 64), lambda i: (0, 0)),
        ],
        out_shape=[
            jax.ShapeDtypeStruct((N_PAD, 64), jnp.float32),
            jax.ShapeDtypeStruct((N_PAD, 128), jnp.float32),
            jax.ShapeDtypeStruct((8, 64), jnp.float32),
        ],
        scratch_shapes=[pltpu.VMEM((8, 64), jnp.float32)],
        compiler_params=_TC_PARAMS,
    )(p0, p1, p2, s0, s1, s2, gp, bp, degp, wc)


def _tc_hop1(acc1, degp):
    def body(a_ref, deg_ref, q1_ref, hs_ref, st_ref, sacc):
        i = pl.program_id(0)
        dinv = _dinv_tile(deg_ref)
        r = (a_ref[0] + a_ref[1]) * dinv
        q1 = r[:, 0:64]
        q1_ref[...] = q1
        hs_ref[...] = jnp.concatenate(
            [r[:, 64:128] * dinv, jnp.zeros((TILE_R, 64), jnp.float32)], axis=1)
        _stats_update(sacc, q1, i)

        @pl.when(i == NT - 1)
        def _():
            st_ref[...] = sacc[...]

    return pl.pallas_call(
        body,
        grid=(NT,),
        in_specs=[
            pl.BlockSpec((2, TILE_R, 128), lambda i: (0, i, 0)),
            pl.BlockSpec((2, TILE_R, 128), lambda i: (0, i, 0)),
        ],
        out_specs=[
            pl.BlockSpec((TILE_R, 64), lambda i: (i, 0)),
            pl.BlockSpec((TILE_R, 128), lambda i: (i, 0)),
            pl.BlockSpec((8, 64), lambda i: (0, 0)),
        ],
        out_shape=[
            jax.ShapeDtypeStruct((N_PAD, 64), jnp.float32),
            jax.ShapeDtypeStruct((N_PAD, 128), jnp.float32),
            jax.ShapeDtypeStruct((8, 64), jnp.float32),
        ],
        scratch_shapes=[pltpu.VMEM((8, 64), jnp.float32)],
        compiler_params=_TC_PARAMS,
    )(acc1, degp)


def _tc_hop2(acc2, degp):
    def body(a_ref, deg_ref, q2_ref, st_ref, sacc):
        i = pl.program_id(0)
        dinv = _dinv_tile(deg_ref)
        q2 = (a_ref[0, :, 0:64] + a_ref[1, :, 0:64]) * dinv
        q2_ref[...] = q2
        _stats_update(sacc, q2, i)

        @pl.when(i == NT - 1)
        def _():
            st_ref[...] = sacc[...]

    return pl.pallas_call(
        body,
        grid=(NT,),
        in_specs=[
            pl.BlockSpec((2, TILE_R, 128), lambda i: (0, i, 0)),
            pl.BlockSpec((2, TILE_R, 128), lambda i: (0, i, 0)),
        ],
        out_specs=[
            pl.BlockSpec((TILE_R, 64), lambda i: (i, 0)),
            pl.BlockSpec((8, 64), lambda i: (0, 0)),
        ],
        out_shape=[
            jax.ShapeDtypeStruct((N_PAD, 64), jnp.float32),
            jax.ShapeDtypeStruct((8, 64), jnp.float32),
        ],
        scratch_shapes=[pltpu.VMEM((8, 64), jnp.float32)],
        compiler_params=_TC_PARAMS,
    )(acc2, degp)


def _tc_out(p0, p1, p2, s0, s1, s2, gp, bp, wout):
    def body(p0_ref, p1_ref, p2_ref, s0_ref, s1_ref, s2_ref, g_ref, b_ref,
             w_ref, y_ref):
        hn = _bn_tanh((p0_ref[...], p1_ref[...], p2_ref[...]),
                      (s0_ref, s1_ref, s2_ref), g_ref, b_ref)
        y_ref[...] = jnp.dot(hn, w_ref[...], preferred_element_type=jnp.float32)

    part = pl.BlockSpec((TILE_R, 64), lambda i: (i, 0))
    st_in = pl.BlockSpec((8, 64), lambda i: (0, 0))
    vec = pl.BlockSpec((1, 192), lambda i: (0, 0))
    return pl.pallas_call(
        body,
        grid=(NT,),
        in_specs=[part, part, part, st_in, st_in, st_in, vec, vec,
                  pl.BlockSpec((192, OUT), lambda i: (0, 0))],
        out_specs=pl.BlockSpec((TILE_R, OUT), lambda i: (i, 0)),
        out_shape=jax.ShapeDtypeStruct((N_PAD, OUT), jnp.float32),
        compiler_params=_TC_PARAMS,
    )(p0, p1, p2, s0, s1, s2, gp, bp, wout)


def _pack_cols(w0, w1, w2):
    z = jnp.zeros((w0.shape[0], 4), jnp.float32)
    return jnp.concatenate([w0, z, w1, z, w2, z], axis=1)


def _pack_rows(w):
    z = jnp.zeros((4, w.shape[1]), jnp.float32)
    return jnp.concatenate(
        [w[0:60], z, w[60:120], z, w[120:180], z], axis=0)


def _pack_vec(v):
    z = jnp.zeros((4,), jnp.float32)
    return jnp.concatenate(
        [v[0:60], z, v[60:120], z, v[120:180], z]).reshape(1, 192)


def kernel(x, edge_index, W1_0, W1_1, W1_2, g1, b1, W2_0, W2_1, W2_2, g2, b2,
           W3_0, W3_1, W3_2, g3, b3, W_out):
    x_pad = jnp.zeros((N_PAD, D_IN), jnp.float32).at[:N].set(x)
    pad = jnp.full((E_PAD - E,), N, jnp.int32)
    srcr = jnp.concatenate([edge_index[0], pad]).reshape(NW, NCH, CH)
    dstr = jnp.concatenate([edge_index[1], pad]).reshape(NW, NCH, CH)

    wc1 = _pack_cols(W1_0, W1_1, W1_2)
    wc2 = _pack_rows(_pack_cols(W2_0, W2_1, W2_2))
    wc3 = _pack_rows(_pack_cols(W3_0, W3_1, W3_2))
    wop = _pack_rows(W_out)

    degp = _propagate(jnp.ones((N_PAD, 128), jnp.float32), srcr, dstr)

    q0, hs, s0 = _tc_first(x_pad, degp, wc1)
    a1 = _propagate(hs, srcr, dstr)
    q1, hs2, s1 = _tc_hop1(a1, degp)
    a2 = _propagate(hs2, srcr, dstr)
    q2, s2 = _tc_hop2(a2, degp)

    for gcur, bcur, wc in ((g1, b1, wc2), (g2, b2, wc3)):
        q0, hs, s0n = _tc_layer(q0, q1, q2, s0, s1, s2,
                                _pack_vec(gcur), _pack_vec(bcur), degp, wc)
        a1 = _propagate(hs, srcr, dstr)
        q1, hs2, s1 = _tc_hop1(a1, degp)
        a2 = _propagate(hs2, srcr, dstr)
        q2, s2 = _tc_hop2(a2, degp)
        s0 = s0n

    y = _tc_out(q0, q1, q2, s0, s1, s2, _pack_vec(g3), _pack_vec(b3), wop)
    return y[:N]


# speedup vs baseline: 1.0800x; 1.0800x over previous
import functools

import jax
import jax.numpy as jnp
from jax import lax
from jax.experimental import pallas as pl
from jax.experimental.pallas import tpu as pltpu
from jax.experimental.pallas import tpu_sc as plsc

N = 10000
N_PAD = 10240
E = 320000
D_IN = 128
HID = 60
OUT = 64
EPS = 1e-5

NC = 2
NS = 16
NW = NC * NS
CH = 128
NCH = 80
IB = 16
NB = NCH // IB
SL = 2
E_PAD = NW * CH * NCH
STRIPE = N_PAD // NS

TILE_R = 1024
NT = N_PAD // TILE_R


def _sc_mesh():
    return plsc.VectorSubcoreMesh(core_axis_name="c", subcore_axis_name="s")


def _fill(rows_ref, value, width):
    vec = jnp.full((16,), value, jnp.float32)

    def body(i, _):
        for k in range(width // 16):
            rows_ref[i, pl.ds(k * 16, 16)] = vec
        return 0

    lax.fori_loop(0, CH, body, 0)


def _propagate(srcr, dstr, hs=None):
    W = 128
    gather = hs is not None
    ins = (hs, srcr, dstr) if gather else (srcr, dstr)
    scratch = [
        pltpu.VMEM((NCH, CH), jnp.int32),
        pltpu.VMEM((CH, W), jnp.float32),
        pltpu.VMEM_SHARED((N_PAD, W), jnp.float32),
        pltpu.SemaphoreType.DMA,
    ]
    if gather:
        scratch.insert(0, pltpu.VMEM((NCH, CH), jnp.int32))

    @functools.partial(
        pl.kernel,
        out_type=jax.ShapeDtypeStruct((NC, N_PAD, W), jnp.float32),
        mesh=_sc_mesh(),
        scratch_types=scratch,
    )
    def k(*refs):
        if gather:
            hs_ref, srcr_ref, dstr_ref, out_ref = refs[:4]
            sidx, didx, rows, acc, gsem = refs[4:]
        else:
            srcr_ref, dstr_ref, out_ref = refs[:3]
            sidx = None
            didx, rows, acc, gsem = refs[3:]
        c = lax.axis_index("c")
        s = lax.axis_index("s")
        wid = c * NS + s
        if gather:
            pltpu.sync_copy(srcr_ref.at[wid], sidx)
        pltpu.sync_copy(dstr_ref.at[wid], didx)
        _fill(rows, 0.0, W)
        for r in range(STRIPE // CH):
            pltpu.sync_copy(rows, acc.at[pl.ds(s * STRIPE + r * CH, CH)])
        if not gather:
            _fill(rows, 1.0, W)
        plsc.subcore_barrier()

        def chunk(j, _):
            if gather:
                pltpu.async_copy(hs_ref.at[sidx.at[j]], rows, gsem).wait()
            pltpu.sync_copy(rows, acc.at[didx.at[j]], add=True)
            return 0

        lax.fori_loop(0, NCH, chunk, 0)
        plsc.subcore_barrier()
        pltpu.sync_copy(acc.at[pl.ds(s * STRIPE, STRIPE)],
                        out_ref.at[c, pl.ds(s * STRIPE, STRIPE)])

    return k(*ins)


_TC_PARAMS = pltpu.CompilerParams(dimension_semantics=("arbitrary",))


def _dinv_tile(deg_ref):
    d = deg_ref[0, :, 0:1] + deg_ref[1, :, 0:1]
    return lax.rsqrt(jnp.maximum(d, 1.0))


def _stats_update(sacc, q, i):
    @pl.when(i == 0)
    def _():
        sacc[...] = jnp.zeros_like(sacc)

    sacc[0:1, :] += jnp.sum(q, axis=0, keepdims=True)
    sacc[1:2, :] += jnp.sum(q * q, axis=0, keepdims=True)


def _tc_first(x, degp, wc):
    def body(x_ref, deg_ref, w_ref, q0_ref, hs_ref, st_ref, sacc):
        i = pl.program_id(0)
        p = jnp.dot(x_ref[...], w_ref[...], preferred_element_type=jnp.float32)
        q0 = p[:, 0:64]
        q0_ref[...] = q0
        dinv = _dinv_tile(deg_ref)
        hs_ref[...] = p[:, 64:192] * dinv
        _stats_update(sacc, q0, i)

        @pl.when(i == NT - 1)
        def _():
            st_ref[...] = sacc[...]

    return pl.pallas_call(
        body,
        grid=(NT,),
        in_specs=[
            pl.BlockSpec((TILE_R, D_IN), lambda i: (i, 0)),
            pl.BlockSpec((2, TILE_R, 128), lambda i: (0, i, 0)),
            pl.BlockSpec((D_IN, 192), lambda i: (0, 0)),
        ],
        out_specs=[
            pl.BlockSpec((TILE_R, 64), lambda i: (i, 0)),
            pl.BlockSpec((TILE_R, 128), lambda i: (i, 0)),
            pl.BlockSpec((8, 64), lambda i: (0, 0)),
        ],
        out_shape=[
            jax.ShapeDtypeStruct((N_PAD, 64), jnp.float32),
            jax.ShapeDtypeStruct((N_PAD, 128), jnp.float32),
            jax.ShapeDtypeStruct((8, 64), jnp.float32),
        ],
        scratch_shapes=[pltpu.VMEM((8, 64), jnp.float32)],
        compiler_params=_TC_PARAMS,
    )(x, degp, wc)


def _bn_tanh(parts, stats, g_ref, b_ref):
    h = jnp.concatenate(parts, axis=1)
    sm = jnp.concatenate([s[0:1, :] for s in stats], axis=1)
    sq = jnp.concatenate([s[1:2, :] for s in stats], axis=1)
    m = sm * (1.0 / N)
    v = sq * (1.0 / N) - m * m
    return jnp.tanh((h - m) * lax.rsqrt(v + EPS) * g_ref[...] + b_ref[...])


def _tc_layer(p0, p1, p2, s0, s1, s2, gp, bp, degp, wc):
    def body(p0_ref, p1_ref, p2_ref, s0_ref, s1_ref, s2_ref, g_ref, b_ref,
             deg_ref, w_ref, q0_ref, hs_ref, st_ref, sacc):
        i = pl.program_id(0)
        hn = _bn_tanh((p0_ref[...], p1_ref[...], p2_ref[...]),
                      (s0_ref, s1_ref, s2_ref), g_ref, b_ref)
        rows = i * TILE_R + lax.broadcasted_iota(jnp.int32, (TILE_R, 1), 0)
        hn = jnp.where(rows < N, hn, 0.0)
        p = jnp.dot(hn, w_ref[...], preferred_element_type=jnp.float32)
        q0 = p[:, 0:64]
        q0_ref[...] = q0
        dinv = _dinv_tile(deg_ref)
        hs_ref[...] = p[:, 64:192] * dinv
        _stats_update(sacc, q0, i)

        @pl.when(i == NT - 1)
        def _():
            st_ref[...] = sacc[...]

    part = pl.BlockSpec((TILE_R, 64), lambda i: (i, 0))
    st_in = pl.BlockSpec((8, 64), lambda i: (0, 0))
    vec = pl.BlockSpec((1, 192), lambda i: (0, 0))
    return pl.pallas_call(
        body,
        grid=(NT,),
        in_specs=[part, part, part, st_in, st_in, st_in, vec, vec,
                  pl.BlockSpec((2, TILE_R, 128), lambda i: (0, i, 0)),
                  pl.BlockSpec((192, 192), lambda i: (0, 0))],
        out_specs=[
            pl.BlockSpec((TILE_R, 64), lambda i: (i, 0)),
            pl.BlockSpec((TILE_R, 128), lambda i: (i, 0)),
            pl.BlockSpec((8, 64), lambda i: (0, 0)),
        ],
        out_shape=[
            jax.ShapeDtypeStruct((N_PAD, 64), jnp.float32),
            jax.ShapeDtypeStruct((N_PAD, 128), jnp.float32),
            jax.ShapeDtypeStruct((8, 64), jnp.float32),
        ],
        scratch_shapes=[pltpu.VMEM((8, 64), jnp.float32)],
        compiler_params=_TC_PARAMS,
    )(p0, p1, p2, s0, s1, s2, gp, bp, degp, wc)


def _tc_hop1(acc1, degp):
    def body(a_ref, deg_ref, q1_ref, hs_ref, st_ref, sacc):
        i = pl.program_id(0)
        dinv = _dinv_tile(deg_ref)
        r = (a_ref[0] + a_ref[1]) * dinv
        q1 = r[:, 0:64]
        q1_ref[...] = q1
        hs_ref[...] = jnp.concatenate(
            [r[:, 64:128] * dinv, jnp.zeros((TILE_R, 64), jnp.float32)], axis=1)
        _stats_update(sacc, q1, i)

        @pl.when(i == NT - 1)
        def _():
            st_ref[...] = sacc[...]

    return pl.pallas_call(
        body,
        grid=(NT,),
        in_specs=[
            pl.BlockSpec((2, TILE_R, 128), lambda i: (0, i, 0)),
            pl.BlockSpec((2, TILE_R, 128), lambda i: (0, i, 0)),
        ],
        out_specs=[
            pl.BlockSpec((TILE_R, 64), lambda i: (i, 0)),
            pl.BlockSpec((TILE_R, 128), lambda i: (i, 0)),
            pl.BlockSpec((8, 64), lambda i: (0, 0)),
        ],
        out_shape=[
            jax.ShapeDtypeStruct((N_PAD, 64), jnp.float32),
            jax.ShapeDtypeStruct((N_PAD, 128), jnp.float32),
            jax.ShapeDtypeStruct((8, 64), jnp.float32),
        ],
        scratch_shapes=[pltpu.VMEM((8, 64), jnp.float32)],
        compiler_params=_TC_PARAMS,
    )(acc1, degp)


def _tc_hop2(acc2, degp):
    def body(a_ref, deg_ref, q2_ref, st_ref, sacc):
        i = pl.program_id(0)
        dinv = _dinv_tile(deg_ref)
        q2 = (a_ref[0, :, 0:64] + a_ref[1, :, 0:64]) * dinv
        q2_ref[...] = q2
        _stats_update(sacc, q2, i)

        @pl.when(i == NT - 1)
        def _():
            st_ref[...] = sacc[...]

    return pl.pallas_call(
        body,
        grid=(NT,),
        in_specs=[
            pl.BlockSpec((2, TILE_R, 128), lambda i: (0, i, 0)),
            pl.BlockSpec((2, TILE_R, 128), lambda i: (0, i, 0)),
        ],
        out_specs=[
            pl.BlockSpec((TILE_R, 64), lambda i: (i, 0)),
            pl.BlockSpec((8, 64), lambda i: (0, 0)),
        ],
        out_shape=[
            jax.ShapeDtypeStruct((N_PAD, 64), jnp.float32),
            jax.ShapeDtypeStruct((8, 64), jnp.float32),
        ],
        scratch_shapes=[pltpu.VMEM((8, 64), jnp.float32)],
        compiler_params=_TC_PARAMS,
    )(acc2, degp)


def _tc_out(p0, p1, p2, s0, s1, s2, gp, bp, wout):
    def body(p0_ref, p1_ref, p2_ref, s0_ref, s1_ref, s2_ref, g_ref, b_ref,
             w_ref, y_ref):
        hn = _bn_tanh((p0_ref[...], p1_ref[...], p2_ref[...]),
                      (s0_ref, s1_ref, s2_ref), g_ref, b_ref)
        y_ref[...] = jnp.dot(hn, w_ref[...], preferred_element_type=jnp.float32)

    part = pl.BlockSpec((TILE_R, 64), lambda i: (i, 0))
    st_in = pl.BlockSpec((8, 64), lambda i: (0, 0))
    vec = pl.BlockSpec((1, 192), lambda i: (0, 0))
    return pl.pallas_call(
        body,
        grid=(NT,),
        in_specs=[part, part, part, st_in, st_in, st_in, vec, vec,
                  pl.BlockSpec((192, OUT), lambda i: (0, 0))],
        out_specs=pl.BlockSpec((TILE_R, OUT), lambda i: (i, 0)),
        out_shape=jax.ShapeDtypeStruct((N_PAD, OUT), jnp.float32),
        compiler_params=_TC_PARAMS,
    )(p0, p1, p2, s0, s1, s2, gp, bp, wout)


def _pack_cols(w0, w1, w2):
    z = jnp.zeros((w0.shape[0], 4), jnp.float32)
    return jnp.concatenate([w0, z, w1, z, w2, z], axis=1)


def _pack_rows(w):
    z = jnp.zeros((4, w.shape[1]), jnp.float32)
    return jnp.concatenate(
        [w[0:60], z, w[60:120], z, w[120:180], z], axis=0)


def _pack_vec(v):
    z = jnp.zeros((4,), jnp.float32)
    return jnp.concatenate(
        [v[0:60], z, v[60:120], z, v[120:180], z]).reshape(1, 192)


def kernel(x, edge_index, W1_0, W1_1, W1_2, g1, b1, W2_0, W2_1, W2_2, g2, b2,
           W3_0, W3_1, W3_2, g3, b3, W_out):
    x_pad = jnp.zeros((N_PAD, D_IN), jnp.float32).at[:N].set(x)
    pad = jnp.full((E_PAD - E,), N, jnp.int32)
    srcr = jnp.concatenate([edge_index[0], pad]).reshape(NW, NCH, CH)
    dstr = jnp.concatenate([edge_index[1], pad]).reshape(NW, NCH, CH)

    wc1 = _pack_cols(W1_0, W1_1, W1_2)
    wc2 = _pack_rows(_pack_cols(W2_0, W2_1, W2_2))
    wc3 = _pack_rows(_pack_cols(W3_0, W3_1, W3_2))
    wop = _pack_rows(W_out)

    degp = _propagate(srcr, dstr)

    q0, hs, s0 = _tc_first(x_pad, degp, wc1)
    a1 = _propagate(srcr, dstr, hs)
    q1, hs2, s1 = _tc_hop1(a1, degp)
    a2 = _propagate(srcr, dstr, hs2)
    q2, s2 = _tc_hop2(a2, degp)

    for gcur, bcur, wc in ((g1, b1, wc2), (g2, b2, wc3)):
        q0, hs, s0n = _tc_layer(q0, q1, q2, s0, s1, s2,
                                _pack_vec(gcur), _pack_vec(bcur), degp, wc)
        a1 = _propagate(srcr, dstr, hs)
        q1, hs2, s1 = _tc_hop1(a1, degp)
        a2 = _propagate(srcr, dstr, hs2)
        q2, s2 = _tc_hop2(a2, degp)
        s0 = s0n

    y = _tc_out(q0, q1, q2, s0, s1, s2, _pack_vec(g3), _pack_vec(b3), wop)
    return y[:N]


# speedup vs baseline: 2.9184x; 2.7022x over previous
import functools

import jax
import jax.numpy as jnp
from jax import lax
from jax.experimental import pallas as pl
from jax.experimental.pallas import tpu as pltpu
from jax.experimental.pallas import tpu_sc as plsc

N = 10000
N_PAD = 10240
E = 320000
D_IN = 128
HID = 60
OUT = 64
EPS = 1e-5

NC = 2
NS = 16
NW = NC * NS
CH = 128
NCH = 80
IB = 16
NB = NCH // IB
SL = 2
E_PAD = NW * CH * NCH
STRIPE = N_PAD // NS

TILE_R = 1024
NT = N_PAD // TILE_R


def _sc_mesh():
    return plsc.VectorSubcoreMesh(core_axis_name="c", subcore_axis_name="s")


def _fill(rows_ref, value, width):
    vec = jnp.full((16,), value, jnp.float32)

    def body(i, _):
        for k in range(width // 16):
            rows_ref[i, pl.ds(k * 16, 16)] = vec
        return 0

    lax.fori_loop(0, CH, body, 0)


def _propagate(srcr, dstr, hs=None):
    W = 128
    gather = hs is not None
    ins = (hs, srcr, dstr) if gather else (srcr, dstr)
    scratch = [
        pltpu.VMEM((NCH, CH), jnp.int32),
        pltpu.VMEM((CH, W), jnp.float32),
        pltpu.VMEM_SHARED((N_PAD, W), jnp.float32),
        pltpu.SemaphoreType.DMA,
    ]
    if gather:
        scratch.insert(0, pltpu.VMEM((NCH, CH), jnp.int32))

    @functools.partial(
        pl.kernel,
        out_type=jax.ShapeDtypeStruct((NC, N_PAD, W), jnp.float32),
        mesh=_sc_mesh(),
        scratch_types=scratch,
    )
    def k(*refs):
        if gather:
            hs_ref, srcr_ref, dstr_ref, out_ref = refs[:4]
            sidx, didx, rows, acc, gsem = refs[4:]
        else:
            srcr_ref, dstr_ref, out_ref = refs[:3]
            sidx = None
            didx, rows, acc, gsem = refs[3:]
        c = lax.axis_index("c")
        s = lax.axis_index("s")
        wid = c * NS + s
        if gather:
            pltpu.sync_copy(srcr_ref.at[wid], sidx)
        pltpu.sync_copy(dstr_ref.at[wid], didx)
        _fill(rows, 0.0, W)
        for r in range(STRIPE // CH):
            pltpu.sync_copy(rows, acc.at[pl.ds(s * STRIPE + r * CH, CH)])
        if not gather:
            _fill(rows, 1.0, W)
        plsc.subcore_barrier()

        def chunk(j, _):
            if gather:
                pltpu.async_copy(hs_ref.at[sidx.at[j]], rows, gsem).wait()
            pltpu.sync_copy(rows, acc.at[didx.at[j]], add=True)
            return 0

        lax.fori_loop(0, NCH, chunk, 0)
        plsc.subcore_barrier()
        pltpu.sync_copy(acc.at[pl.ds(s * STRIPE, STRIPE)],
                        out_ref.at[c, pl.ds(s * STRIPE, STRIPE)])

    return k(*ins)


_TC_PARAMS = pltpu.CompilerParams(dimension_semantics=("arbitrary",))


def _dinv_tile(deg_ref):
    d = deg_ref[0, :, 0:1] + deg_ref[1, :, 0:1]
    return lax.rsqrt(jnp.maximum(d, 1.0))


def _stats_update(sacc, q, i):
    @pl.when(i == 0)
    def _():
        sacc[...] = jnp.zeros_like(sacc)

    sacc[0:1, :] += jnp.sum(q, axis=0, keepdims=True)
    sacc[1:2, :] += jnp.sum(q * q, axis=0, keepdims=True)


def _tc_first(x, degp, wc):
    def body(x_ref, deg_ref, w_ref, q0_ref, hs_ref, st_ref, sacc):
        i = pl.program_id(0)
        p = jnp.dot(x_ref[...], w_ref[...], preferred_element_type=jnp.float32)
        q0 = p[:, 0:64]
        q0_ref[...] = q0
        dinv = _dinv_tile(deg_ref)
        hs_ref[...] = p[:, 64:192] * dinv
        _stats_update(sacc, q0, i)

        @pl.when(i == NT - 1)
        def _():
            st_ref[...] = sacc[...]

    return pl.pallas_call(
        body,
        grid=(NT,),
        in_specs=[
            pl.BlockSpec((TILE_R, D_IN), lambda i: (i, 0)),
            pl.BlockSpec((2, TILE_R, 128), lambda i: (0, i, 0)),
            pl.BlockSpec((D_IN, 192), lambda i: (0, 0)),
        ],
        out_specs=[
            pl.BlockSpec((TILE_R, 64), lambda i: (i, 0)),
            pl.BlockSpec((TILE_R, 128), lambda i: (i, 0)),
            pl.BlockSpec((8, 64), lambda i: (0, 0)),
        ],
        out_shape=[
            jax.ShapeDtypeStruct((N_PAD, 64), jnp.float32),
            jax.ShapeDtypeStruct((N_PAD, 128), jnp.float32),
            jax.ShapeDtypeStruct((8, 64), jnp.float32),
        ],
        scratch_shapes=[pltpu.VMEM((8, 64), jnp.float32)],
        compiler_params=_TC_PARAMS,
    )(x, degp, wc)


def _bn_tanh(parts, stats, g_ref, b_ref):
    h = jnp.concatenate(parts, axis=1)
    sm = jnp.concatenate([s[0:1, :] for s in stats], axis=1)
    sq = jnp.concatenate([s[1:2, :] for s in stats], axis=1)
    m = sm * (1.0 / N)
    v = sq * (1.0 / N) - m * m
    return jnp.tanh((h - m) * lax.rsqrt(v + EPS) * g_ref[...] + b_ref[...])


def _tc_layer(p0, p1, p2, s0, s1, s2, gp, bp, degp, wc):
    def body(p0_ref, p1_ref, p2_ref, s0_ref, s1_ref, s2_ref, g_ref, b_ref,
             deg_ref, w_ref, q0_ref, hs_ref, st_ref, sacc):
        i = pl.program_id(0)
        hn = _bn_tanh((p0_ref[...], p1_ref[...], p2_ref[...]),
                      (s0_ref, s1_ref, s2_ref), g_ref, b_ref)
        rows = i * TILE_R + lax.broadcasted_iota(jnp.int32, (TILE_R, 1), 0)
        hn = jnp.where(rows < N, hn, 0.0)
        p = jnp.dot(hn, w_ref[...], preferred_element_type=jnp.float32)
        q0 = p[:, 0:64]
        q0_ref[...] = q0
        dinv = _dinv_tile(deg_ref)
        hs_ref[...] = p[:, 64:192] * dinv
        _stats_update(sacc, q0, i)

        @pl.when(i == NT - 1)
        def _():
            st_ref[...] = sacc[...]

    part = pl.BlockSpec((TILE_R, 64), lambda i: (i, 0))
    st_in = pl.BlockSpec((8, 64), lambda i: (0, 0))
    vec = pl.BlockSpec((1, 192), lambda i: (0, 0))
    return pl.pallas_call(
        body,
        grid=(NT,),
        in_specs=[part, part, part, st_in, st_in, st_in, vec, vec,
                  pl.BlockSpec((2, TILE_R, 128), lambda i: (0, i, 0)),
                  pl.BlockSpec((192, 192), lambda i: (0, 0))],
        out_specs=[
            pl.BlockSpec((TILE_R, 64), lambda i: (i, 0)),
            pl.BlockSpec((TILE_R, 128), lambda i: (i, 0)),
            pl.BlockSpec((8, 64), lambda i: (0, 0)),
        ],
        out_shape=[
            jax.ShapeDtypeStruct((N_PAD, 64), jnp.float32),
            jax.ShapeDtypeStruct((N_PAD, 128), jnp.float32),
            jax.ShapeDtypeStruct((8, 64), jnp.float32),
        ],
        scratch_shapes=[pltpu.VMEM((8, 64), jnp.float32)],
        compiler_params=_TC_PARAMS,
    )(p0, p1, p2, s0, s1, s2, gp, bp, degp, wc)


def _tc_hop1(acc1, degp):
    def body(a_ref, deg_ref, q1_ref, hs_ref, st_ref, sacc):
        i = pl.program_id(0)
        dinv = _dinv_tile(deg_ref)
        r = (a_ref[0] + a_ref[1]) * dinv
        q1 = r[:, 0:64]
        q1_ref[...] = q1
        hs_ref[...] = jnp.concatenate(
            [r[:, 64:128] * dinv, jnp.zeros((TILE_R, 64), jnp.float32)], axis=1)
        _stats_update(sacc, q1, i)

        @pl.when(i == NT - 1)
        def _():
            st_ref[...] = sacc[...]

    return pl.pallas_call(
        body,
        grid=(NT,),
        in_specs=[
            pl.BlockSpec((2, TILE_R, 128), lambda i: (0, i, 0)),
            pl.BlockSpec((2, TILE_R, 128), lambda i: (0, i, 0)),
        ],
        out_specs=[
            pl.BlockSpec((TILE_R, 64), lambda i: (i, 0)),
            pl.BlockSpec((TILE_R, 128), lambda i: (i, 0)),
            pl.BlockSpec((8, 64), lambda i: (0, 0)),
        ],
        out_shape=[
            jax.ShapeDtypeStruct((N_PAD, 64), jnp.float32),
            jax.ShapeDtypeStruct((N_PAD, 128), jnp.float32),
            jax.ShapeDtypeStruct((8, 64), jnp.float32),
        ],
        scratch_shapes=[pltpu.VMEM((8, 64), jnp.float32)],
        compiler_params=_TC_PARAMS,
    )(acc1, degp)


def _tc_hop2(acc2, degp):
    def body(a_ref, deg_ref, q2_ref, st_ref, sacc):
        i = pl.program_id(0)
        dinv = _dinv_tile(deg_ref)
        q2 = (a_ref[0, :, 0:64] + a_ref[1, :, 0:64]) * dinv
        q2_ref[...] = q2
        _stats_update(sacc, q2, i)

        @pl.when(i == NT - 1)
        def _():
            st_ref[...] = sacc[...]

    return pl.pallas_call(
        body,
        grid=(NT,),
        in_specs=[
            pl.BlockSpec((2, TILE_R, 128), lambda i: (0, i, 0)),
            pl.BlockSpec((2, TILE_R, 128), lambda i: (0, i, 0)),
        ],
        out_specs=[
            pl.BlockSpec((TILE_R, 64), lambda i: (i, 0)),
            pl.BlockSpec((8, 64), lambda i: (0, 0)),
        ],
        out_shape=[
            jax.ShapeDtypeStruct((N_PAD, 64), jnp.float32),
            jax.ShapeDtypeStruct((8, 64), jnp.float32),
        ],
        scratch_shapes=[pltpu.VMEM((8, 64), jnp.float32)],
        compiler_params=_TC_PARAMS,
    )(acc2, degp)


def _tc_out(p0, p1, p2, s0, s1, s2, gp, bp, wout):
    def body(p0_ref, p1_ref, p2_ref, s0_ref, s1_ref, s2_ref, g_ref, b_ref,
             w_ref, y_ref):
        hn = _bn_tanh((p0_ref[...], p1_ref[...], p2_ref[...]),
                      (s0_ref, s1_ref, s2_ref), g_ref, b_ref)
        y_ref[...] = jnp.dot(hn, w_ref[...], preferred_element_type=jnp.float32)

    part = pl.BlockSpec((TILE_R, 64), lambda i: (i, 0))
    st_in = pl.BlockSpec((8, 64), lambda i: (0, 0))
    vec = pl.BlockSpec((1, 192), lambda i: (0, 0))
    return pl.pallas_call(
        body,
        grid=(NT,),
        in_specs=[part, part, part, st_in, st_in, st_in, vec, vec,
                  pl.BlockSpec((192, OUT), lambda i: (0, 0))],
        out_specs=pl.BlockSpec((TILE_R, OUT), lambda i: (i, 0)),
        out_shape=jax.ShapeDtypeStruct((N_PAD, OUT), jnp.float32),
        compiler_params=_TC_PARAMS,
    )(p0, p1, p2, s0, s1, s2, gp, bp, wout)


def _pack_cols(w0, w1, w2):
    z = jnp.zeros((w0.shape[0], 4), jnp.float32)
    return jnp.concatenate([w0, z, w1, z, w2, z], axis=1)


def _pack_rows(w):
    z = jnp.zeros((4, w.shape[1]), jnp.float32)
    return jnp.concatenate(
        [w[0:60], z, w[60:120], z, w[120:180], z], axis=0)


def _pack_vec(v):
    z = jnp.zeros((4,), jnp.float32)
    return jnp.concatenate(
        [v[0:60], z, v[60:120], z, v[120:180], z]).reshape(1, 192)


def kernel(x, edge_index, W1_0, W1_1, W1_2, g1, b1, W2_0, W2_1, W2_2, g2, b2,
           W3_0, W3_1, W3_2, g3, b3, W_out):
    x_pad = jnp.zeros((N_PAD, D_IN), jnp.float32).at[:N].set(x)
    pad = N + (jnp.arange(E_PAD - E, dtype=jnp.int32) % (N_PAD - N))
    srcr = jnp.concatenate([edge_index[0], pad]).reshape(NW, NCH, CH)
    dstr = jnp.concatenate([edge_index[1], pad]).reshape(NW, NCH, CH)

    wc1 = _pack_cols(W1_0, W1_1, W1_2)
    wc2 = _pack_rows(_pack_cols(W2_0, W2_1, W2_2))
    wc3 = _pack_rows(_pack_cols(W3_0, W3_1, W3_2))
    wop = _pack_rows(W_out)

    degp = _propagate(srcr, dstr)

    q0, hs, s0 = _tc_first(x_pad, degp, wc1)
    a1 = _propagate(srcr, dstr, hs)
    q1, hs2, s1 = _tc_hop1(a1, degp)
    a2 = _propagate(srcr, dstr, hs2)
    q2, s2 = _tc_hop2(a2, degp)

    for gcur, bcur, wc in ((g1, b1, wc2), (g2, b2, wc3)):
        q0, hs, s0n = _tc_layer(q0, q1, q2, s0, s1, s2,
                                _pack_vec(gcur), _pack_vec(bcur), degp, wc)
        a1 = _propagate(srcr, dstr, hs)
        q1, hs2, s1 = _tc_hop1(a1, degp)
        a2 = _propagate(srcr, dstr, hs2)
        q2, s2 = _tc_hop2(a2, degp)
        s0 = s0n

    y = _tc_out(q0, q1, q2, s0, s1, s2, _pack_vec(g3), _pack_vec(b3), wop)
    return y[:N]


# speedup vs baseline: 4.1168x; 1.4106x over previous
import functools

import jax
import jax.numpy as jnp
from jax import lax
from jax.experimental import pallas as pl
from jax.experimental.pallas import tpu as pltpu
from jax.experimental.pallas import tpu_sc as plsc

N = 10000
N_PAD = 10240
E = 320000
D_IN = 128
HID = 60
OUT = 64
EPS = 1e-5

NC = 2
NS = 16
NW = NC * NS
CH = 128
NCH = 80
IB = 16
NB = NCH // IB
SL = 2
E_PAD = NW * CH * NCH
STRIPE = N_PAD // NS

TILE_R = 1024
NT = N_PAD // TILE_R


def _sc_mesh():
    return plsc.VectorSubcoreMesh(core_axis_name="c", subcore_axis_name="s")


def _fill(rows_ref, value, width):
    vec = jnp.full((16,), value, jnp.float32)

    def body(i, _):
        for k in range(width // 16):
            rows_ref[i, pl.ds(k * 16, 16)] = vec
        return 0

    lax.fori_loop(0, CH, body, 0)


def _propagate(srcr, dstr, hs=None):
    W = 128
    gather = hs is not None
    ins = (hs, srcr, dstr) if gather else (srcr, dstr)
    if gather:
        scratch = [
            pltpu.VMEM((2, IB, CH), jnp.int32),
            pltpu.VMEM((NCH, CH), jnp.int32),
            pltpu.VMEM((CH, W), jnp.float32),
            pltpu.VMEM((CH, W), jnp.float32),
            pltpu.VMEM_SHARED((N_PAD, W), jnp.float32),
            pltpu.SemaphoreType.DMA,
            pltpu.SemaphoreType.DMA,
            pltpu.SemaphoreType.DMA,
            pltpu.SemaphoreType.DMA,
        ]
    else:
        scratch = [
            pltpu.VMEM((NCH, CH), jnp.int32),
            pltpu.VMEM((CH, W), jnp.float32),
            pltpu.VMEM_SHARED((N_PAD, W), jnp.float32),
            pltpu.SemaphoreType.DMA,
        ]

    @functools.partial(
        pl.kernel,
        out_type=jax.ShapeDtypeStruct((NC, N_PAD, W), jnp.float32),
        mesh=_sc_mesh(),
        scratch_types=scratch,
    )
    def k(*refs):
        if gather:
            hs_ref, srcr_ref, dstr_ref, out_ref = refs[:4]
            sidxb, didx, rows0, rows1, acc, gsem0, gsem1, isem0, isem1 = refs[4:]
            rows = rows0
        else:
            srcr_ref, dstr_ref, out_ref = refs[:3]
            didx, rows, acc, gsem = refs[3:]
        c = lax.axis_index("c")
        s = lax.axis_index("s")
        wid = c * NS + s

        if gather:
            isem = (isem0, isem1)

            def sidx_fetch(kb, slot):
                pltpu.async_copy(srcr_ref.at[wid, pl.ds(kb * IB, IB)],
                                 sidxb.at[slot], isem[slot])

            sidx_fetch(0, 0)
        pltpu.sync_copy(dstr_ref.at[wid], didx)
        _fill(rows, 0.0, W)
        for r in range(STRIPE // CH):
            pltpu.sync_copy(rows, acc.at[pl.ds(s * STRIPE + r * CH, CH)])
        if not gather:
            _fill(rows, 1.0, W)
        plsc.subcore_barrier()

        if gather:
            rbuf = (rows0, rows1)
            gsem = (gsem0, gsem1)

            def gath(slot, l, b):
                pltpu.async_copy(hs_ref.at[sidxb.at[slot, l]], rbuf[b],
                                 gsem[b])

            def wait_g(b):
                pltpu.make_async_copy(hs_ref.at[sidxb.at[0, 0]], rbuf[b],
                                      gsem[b]).wait()

            def scat(j, b):
                pltpu.sync_copy(rbuf[b], acc.at[didx.at[j]], add=True)

            for kb in range(NB):
                slot = kb % 2
                pltpu.make_async_copy(srcr_ref.at[wid, pl.ds(0, IB)],
                                      sidxb.at[slot], isem[slot]).wait()
                if kb + 1 < NB:
                    sidx_fetch(kb + 1, 1 - slot)
                base = kb * IB
                gath(slot, 0, 0)
                gath(slot, 1, 1)

                def pair(a, _):
                    wait_g(0)
                    scat(base + 2 * a, 0)
                    gath(slot, 2 * a + 2, 0)
                    wait_g(1)
                    scat(base + 2 * a + 1, 1)
                    gath(slot, 2 * a + 3, 1)
                    return 0

                lax.fori_loop(0, IB // 2 - 1, pair, 0)
                wait_g(0)
                scat(base + IB - 2, 0)
                wait_g(1)
                scat(base + IB - 1, 1)
        else:
            def chunk(j, _):
                pltpu.sync_copy(rows, acc.at[didx.at[j]], add=True)
                return 0

            lax.fori_loop(0, NCH, chunk, 0)

        plsc.subcore_barrier()
        pltpu.sync_copy(acc.at[pl.ds(s * STRIPE, STRIPE)],
                        out_ref.at[c, pl.ds(s * STRIPE, STRIPE)])

    return k(*ins)


_TC_PARAMS = pltpu.CompilerParams(dimension_semantics=("arbitrary",))


def _dinv_tile(deg_ref):
    d = deg_ref[0, :, 0:1] + deg_ref[1, :, 0:1]
    return lax.rsqrt(jnp.maximum(d, 1.0))


def _stats_update(sacc, q, i):
    @pl.when(i == 0)
    def _():
        sacc[...] = jnp.zeros_like(sacc)

    sacc[0:1, :] += jnp.sum(q, axis=0, keepdims=True)
    sacc[1:2, :] += jnp.sum(q * q, axis=0, keepdims=True)


def _tc_first(x, degp, wc):
    def body(x_ref, deg_ref, w_ref, q0_ref, hs_ref, st_ref, sacc):
        i = pl.program_id(0)
        p = jnp.dot(x_ref[...], w_ref[...], preferred_element_type=jnp.float32)
        q0 = p[:, 0:64]
        q0_ref[...] = q0
        dinv = _dinv_tile(deg_ref)
        hs_ref[...] = p[:, 64:192] * dinv
        _stats_update(sacc, q0, i)

        @pl.when(i == NT - 1)
        def _():
            st_ref[...] = sacc[...]

    return pl.pallas_call(
        body,
        grid=(NT,),
        in_specs=[
            pl.BlockSpec((TILE_R, D_IN), lambda i: (i, 0)),
            pl.BlockSpec((2, TILE_R, 128), lambda i: (0, i, 0)),
            pl.BlockSpec((D_IN, 192), lambda i: (0, 0)),
        ],
        out_specs=[
            pl.BlockSpec((TILE_R, 64), lambda i: (i, 0)),
            pl.BlockSpec((TILE_R, 128), lambda i: (i, 0)),
            pl.BlockSpec((8, 64), lambda i: (0, 0)),
        ],
        out_shape=[
            jax.ShapeDtypeStruct((N_PAD, 64), jnp.float32),
            jax.ShapeDtypeStruct((N_PAD, 128), jnp.float32),
            jax.ShapeDtypeStruct((8, 64), jnp.float32),
        ],
        scratch_shapes=[pltpu.VMEM((8, 64), jnp.float32)],
        compiler_params=_TC_PARAMS,
    )(x, degp, wc)


def _bn_tanh(parts, stats, g_ref, b_ref):
    h = jnp.concatenate(parts, axis=1)
    sm = jnp.concatenate([s[0:1, :] for s in stats], axis=1)
    sq = jnp.concatenate([s[1:2, :] for s in stats], axis=1)
    m = sm * (1.0 / N)
    v = sq * (1.0 / N) - m * m
    return jnp.tanh((h - m) * lax.rsqrt(v + EPS) * g_ref[...] + b_ref[...])


def _tc_layer(p0, p1, p2, s0, s1, s2, gp, bp, degp, wc):
    def body(p0_ref, p1_ref, p2_ref, s0_ref, s1_ref, s2_ref, g_ref, b_ref,
             deg_ref, w_ref, q0_ref, hs_ref, st_ref, sacc):
        i = pl.program_id(0)
        hn = _bn_tanh((p0_ref[...], p1_ref[...], p2_ref[...]),
                      (s0_ref, s1_ref, s2_ref), g_ref, b_ref)
        rows = i * TILE_R + lax.broadcasted_iota(jnp.int32, (TILE_R, 1), 0)
        hn = jnp.where(rows < N, hn, 0.0)
        p = jnp.dot(hn, w_ref[...], preferred_element_type=jnp.float32)
        q0 = p[:, 0:64]
        q0_ref[...] = q0
        dinv = _dinv_tile(deg_ref)
        hs_ref[...] = p[:, 64:192] * dinv
        _stats_update(sacc, q0, i)

        @pl.when(i == NT - 1)
        def _():
            st_ref[...] = sacc[...]

    part = pl.BlockSpec((TILE_R, 64), lambda i: (i, 0))
    st_in = pl.BlockSpec((8, 64), lambda i: (0, 0))
    vec = pl.BlockSpec((1, 192), lambda i: (0, 0))
    return pl.pallas_call(
        body,
        grid=(NT,),
        in_specs=[part, part, part, st_in, st_in, st_in, vec, vec,
                  pl.BlockSpec((2, TILE_R, 128), lambda i: (0, i, 0)),
                  pl.BlockSpec((192, 192), lambda i: (0, 0))],
        out_specs=[
            pl.BlockSpec((TILE_R, 64), lambda i: (i, 0)),
            pl.BlockSpec((TILE_R, 128), lambda i: (i, 0)),
            pl.BlockSpec((8, 64), lambda i: (0, 0)),
        ],
        out_shape=[
            jax.ShapeDtypeStruct((N_PAD, 64), jnp.float32),
            jax.ShapeDtypeStruct((N_PAD, 128), jnp.float32),
            jax.ShapeDtypeStruct((8, 64), jnp.float32),
        ],
        scratch_shapes=[pltpu.VMEM((8, 64), jnp.float32)],
        compiler_params=_TC_PARAMS,
    )(p0, p1, p2, s0, s1, s2, gp, bp, degp, wc)


def _tc_hop1(acc1, degp):
    def body(a_ref, deg_ref, q1_ref, hs_ref, st_ref, sacc):
        i = pl.program_id(0)
        dinv = _dinv_tile(deg_ref)
        r = (a_ref[0] + a_ref[1]) * dinv
        q1 = r[:, 0:64]
        q1_ref[...] = q1
        hs_ref[...] = jnp.concatenate(
            [r[:, 64:128] * dinv, jnp.zeros((TILE_R, 64), jnp.float32)], axis=1)
        _stats_update(sacc, q1, i)

        @pl.when(i == NT - 1)
        def _():
            st_ref[...] = sacc[...]

    return pl.pallas_call(
        body,
        grid=(NT,),
        in_specs=[
            pl.BlockSpec((2, TILE_R, 128), lambda i: (0, i, 0)),
            pl.BlockSpec((2, TILE_R, 128), lambda i: (0, i, 0)),
        ],
        out_specs=[
            pl.BlockSpec((TILE_R, 64), lambda i: (i, 0)),
            pl.BlockSpec((TILE_R, 128), lambda i: (i, 0)),
            pl.BlockSpec((8, 64), lambda i: (0, 0)),
        ],
        out_shape=[
            jax.ShapeDtypeStruct((N_PAD, 64), jnp.float32),
            jax.ShapeDtypeStruct((N_PAD, 128), jnp.float32),
            jax.ShapeDtypeStruct((8, 64), jnp.float32),
        ],
        scratch_shapes=[pltpu.VMEM((8, 64), jnp.float32)],
        compiler_params=_TC_PARAMS,
    )(acc1, degp)


def _tc_hop2(acc2, degp):
    def body(a_ref, deg_ref, q2_ref, st_ref, sacc):
        i = pl.program_id(0)
        dinv = _dinv_tile(deg_ref)
        q2 = (a_ref[0, :, 0:64] + a_ref[1, :, 0:64]) * dinv
        q2_ref[...] = q2
        _stats_update(sacc, q2, i)

        @pl.when(i == NT - 1)
        def _():
            st_ref[...] = sacc[...]

    return pl.pallas_call(
        body,
        grid=(NT,),
        in_specs=[
            pl.BlockSpec((2, TILE_R, 128), lambda i: (0, i, 0)),
            pl.BlockSpec((2, TILE_R, 128), lambda i: (0, i, 0)),
        ],
        out_specs=[
            pl.BlockSpec((TILE_R, 64), lambda i: (i, 0)),
            pl.BlockSpec((8, 64), lambda i: (0, 0)),
        ],
        out_shape=[
            jax.ShapeDtypeStruct((N_PAD, 64), jnp.float32),
            jax.ShapeDtypeStruct((8, 64), jnp.float32),
        ],
        scratch_shapes=[pltpu.VMEM((8, 64), jnp.float32)],
        compiler_params=_TC_PARAMS,
    )(acc2, degp)


def _tc_out(p0, p1, p2, s0, s1, s2, gp, bp, wout):
    def body(p0_ref, p1_ref, p2_ref, s0_ref, s1_ref, s2_ref, g_ref, b_ref,
             w_ref, y_ref):
        hn = _bn_tanh((p0_ref[...], p1_ref[...], p2_ref[...]),
                      (s0_ref, s1_ref, s2_ref), g_ref, b_ref)
        y_ref[...] = jnp.dot(hn, w_ref[...], preferred_element_type=jnp.float32)

    part = pl.BlockSpec((TILE_R, 64), lambda i: (i, 0))
    st_in = pl.BlockSpec((8, 64), lambda i: (0, 0))
    vec = pl.BlockSpec((1, 192), lambda i: (0, 0))
    return pl.pallas_call(
        body,
        grid=(NT,),
        in_specs=[part, part, part, st_in, st_in, st_in, vec, vec,
                  pl.BlockSpec((192, OUT), lambda i: (0, 0))],
        out_specs=pl.BlockSpec((TILE_R, OUT), lambda i: (i, 0)),
        out_shape=jax.ShapeDtypeStruct((N_PAD, OUT), jnp.float32),
        compiler_params=_TC_PARAMS,
    )(p0, p1, p2, s0, s1, s2, gp, bp, wout)


def _pack_cols(w0, w1, w2):
    z = jnp.zeros((w0.shape[0], 4), jnp.float32)
    return jnp.concatenate([w0, z, w1, z, w2, z], axis=1)


def _pack_rows(w):
    z = jnp.zeros((4, w.shape[1]), jnp.float32)
    return jnp.concatenate(
        [w[0:60], z, w[60:120], z, w[120:180], z], axis=0)


def _pack_vec(v):
    z = jnp.zeros((4,), jnp.float32)
    return jnp.concatenate(
        [v[0:60], z, v[60:120], z, v[120:180], z]).reshape(1, 192)


def kernel(x, edge_index, W1_0, W1_1, W1_2, g1, b1, W2_0, W2_1, W2_2, g2, b2,
           W3_0, W3_1, W3_2, g3, b3, W_out):
    x_pad = jnp.zeros((N_PAD, D_IN), jnp.float32).at[:N].set(x)
    pad = N + (jnp.arange(E_PAD - E, dtype=jnp.int32) % (N_PAD - N))
    srcr = jnp.concatenate([edge_index[0], pad]).reshape(NW, NCH, CH)
    dstr = jnp.concatenate([edge_index[1], pad]).reshape(NW, NCH, CH)

    wc1 = _pack_cols(W1_0, W1_1, W1_2)
    wc2 = _pack_rows(_pack_cols(W2_0, W2_1, W2_2))
    wc3 = _pack_rows(_pack_cols(W3_0, W3_1, W3_2))
    wop = _pack_rows(W_out)

    degp = _propagate(srcr, dstr)

    q0, hs, s0 = _tc_first(x_pad, degp, wc1)
    a1 = _propagate(srcr, dstr, hs)
    q1, hs2, s1 = _tc_hop1(a1, degp)
    a2 = _propagate(srcr, dstr, hs2)
    q2, s2 = _tc_hop2(a2, degp)

    for gcur, bcur, wc in ((g1, b1, wc2), (g2, b2, wc3)):
        q0, hs, s0n = _tc_layer(q0, q1, q2, s0, s1, s2,
                                _pack_vec(gcur), _pack_vec(bcur), degp, wc)
        a1 = _propagate(srcr, dstr, hs)
        q1, hs2, s1 = _tc_hop1(a1, degp)
        a2 = _propagate(srcr, dstr, hs2)
        q2, s2 = _tc_hop2(a2, degp)
        s0 = s0n

    y = _tc_out(q0, q1, q2, s0, s1, s2, _pack_vec(g3), _pack_vec(b3), wop)
    return y[:N]


# speedup vs baseline: 4.1295x; 1.0031x over previous
import functools

import jax
import jax.numpy as jnp
from jax import lax
from jax.experimental import pallas as pl
from jax.experimental.pallas import tpu as pltpu
from jax.experimental.pallas import tpu_sc as plsc

N = 10000
N_PAD = 10240
E = 320000
D_IN = 128
HID = 60
OUT = 64
EPS = 1e-5

NC = 2
NS = 16
NW = NC * NS
CH = 128
NCH = 80
IB = 16
NB = NCH // IB
SL = 2
E_PAD = NW * CH * NCH
STRIPE = N_PAD // NS

TILE_R = 1024
NT = N_PAD // TILE_R


def _sc_mesh():
    return plsc.VectorSubcoreMesh(core_axis_name="c", subcore_axis_name="s")


def _fill(rows_ref, value, width):
    vec = jnp.full((16,), value, jnp.float32)

    def body(i, _):
        for k in range(width // 16):
            rows_ref[i, pl.ds(k * 16, 16)] = vec
        return 0

    lax.fori_loop(0, CH, body, 0)


def _propagate(srcr, dstr, hs=None):
    W = 128
    gather = hs is not None
    ins = (hs, srcr, dstr) if gather else (srcr, dstr)
    if gather:
        scratch = [
            pltpu.VMEM((2, IB, CH), jnp.int32),
            pltpu.VMEM((NCH, CH), jnp.int32),
            pltpu.VMEM((CH, W), jnp.float32),
            pltpu.VMEM((CH, W), jnp.float32),
            pltpu.VMEM_SHARED((N_PAD, W), jnp.float32),
            pltpu.SemaphoreType.DMA,
            pltpu.SemaphoreType.DMA,
            pltpu.SemaphoreType.DMA,
            pltpu.SemaphoreType.DMA,
        ]
    else:
        scratch = [
            pltpu.VMEM((NCH, CH), jnp.int32),
            pltpu.VMEM((CH, W), jnp.float32),
            pltpu.VMEM_SHARED((N_PAD, W), jnp.float32),
            pltpu.SemaphoreType.DMA,
            pltpu.SemaphoreType.DMA,
        ]

    @functools.partial(
        pl.kernel,
        out_type=jax.ShapeDtypeStruct((NC, N_PAD, W), jnp.float32),
        mesh=_sc_mesh(),
        scratch_types=scratch,
    )
    def k(*refs):
        if gather:
            hs_ref, srcr_ref, dstr_ref, out_ref = refs[:4]
            sidxb, didx, rows0, rows1, acc, gsem0, gsem1, isem0, isem1 = refs[4:]
            rows = rows0
        else:
            srcr_ref, dstr_ref, out_ref = refs[:3]
            didx, rows, acc, ssem0, ssem1 = refs[3:]
        c = lax.axis_index("c")
        s = lax.axis_index("s")
        wid = c * NS + s

        if gather:
            isem = (isem0, isem1)

            def sidx_fetch(kb, slot):
                pltpu.async_copy(srcr_ref.at[wid, pl.ds(kb * IB, IB)],
                                 sidxb.at[slot], isem[slot])

            sidx_fetch(0, 0)
        pltpu.sync_copy(dstr_ref.at[wid], didx)
        _fill(rows, 0.0, W)
        for r in range(STRIPE // CH):
            pltpu.sync_copy(rows, acc.at[pl.ds(s * STRIPE + r * CH, CH)])
        if not gather:
            _fill(rows, 1.0, W)
        plsc.subcore_barrier()

        if gather:
            rbuf = (rows0, rows1)
            gsem = (gsem0, gsem1)

            def gath(slot, l, b):
                pltpu.async_copy(hs_ref.at[sidxb.at[slot, l]], rbuf[b],
                                 gsem[b])

            def wait_g(b):
                pltpu.make_async_copy(hs_ref.at[sidxb.at[0, 0]], rbuf[b],
                                      gsem[b]).wait()

            def scat(j, b):
                pltpu.sync_copy(rbuf[b], acc.at[didx.at[j]], add=True)

            for kb in range(NB):
                slot = kb % 2
                pltpu.make_async_copy(srcr_ref.at[wid, pl.ds(0, IB)],
                                      sidxb.at[slot], isem[slot]).wait()
                if kb + 1 < NB:
                    sidx_fetch(kb + 1, 1 - slot)
                base = kb * IB
                gath(slot, 0, 0)
                gath(slot, 1, 1)

                def pair(a, _):
                    wait_g(0)
                    scat(base + 2 * a, 0)
                    gath(slot, 2 * a + 2, 0)
                    wait_g(1)
                    scat(base + 2 * a + 1, 1)
                    gath(slot, 2 * a + 3, 1)
                    return 0

                lax.fori_loop(0, IB // 2 - 1, pair, 0)
                wait_g(0)
                scat(base + IB - 2, 0)
                wait_g(1)
                scat(base + IB - 1, 1)
        else:
            ssem = (ssem0, ssem1)

            def scat_a(j, b):
                pltpu.async_copy(rows, acc.at[didx.at[j]], ssem[b], add=True)

            def wait_s(b):
                pltpu.make_async_copy(rows, acc.at[didx.at[0]],
                                      ssem[b]).wait()

            scat_a(0, 0)
            scat_a(1, 1)

            def chunk(a, _):
                wait_s(0)
                scat_a(2 * a + 2, 0)
                wait_s(1)
                scat_a(2 * a + 3, 1)
                return 0

            lax.fori_loop(0, NCH // 2 - 1, chunk, 0)
            wait_s(0)
            wait_s(1)

        plsc.subcore_barrier()
        pltpu.sync_copy(acc.at[pl.ds(s * STRIPE, STRIPE)],
                        out_ref.at[c, pl.ds(s * STRIPE, STRIPE)])

    return k(*ins)


_TC_PARAMS = pltpu.CompilerParams(dimension_semantics=("arbitrary",))


def _dinv_tile(deg_ref):
    d = deg_ref[0, :, 0:1] + deg_ref[1, :, 0:1]
    return lax.rsqrt(jnp.maximum(d, 1.0))


def _stats_update(sacc, q, i):
    @pl.when(i == 0)
    def _():
        sacc[...] = jnp.zeros_like(sacc)

    sacc[0:1, :] += jnp.sum(q, axis=0, keepdims=True)
    sacc[1:2, :] += jnp.sum(q * q, axis=0, keepdims=True)


def _tc_first(x, degp, wc):
    def body(x_ref, deg_ref, w_ref, q0_ref, hs_ref, st_ref, sacc):
        i = pl.program_id(0)
        p = jnp.dot(x_ref[...], w_ref[...], preferred_element_type=jnp.float32)
        q0 = p[:, 0:64]
        q0_ref[...] = q0
        dinv = _dinv_tile(deg_ref)
        hs_ref[...] = p[:, 64:192] * dinv
        _stats_update(sacc, q0, i)

        @pl.when(i == NT - 1)
        def _():
            st_ref[...] = sacc[...]

    return pl.pallas_call(
        body,
        grid=(NT,),
        in_specs=[
            pl.BlockSpec((TILE_R, D_IN), lambda i: (i, 0)),
            pl.BlockSpec((2, TILE_R, 128), lambda i: (0, i, 0)),
            pl.BlockSpec((D_IN, 192), lambda i: (0, 0)),
        ],
        out_specs=[
            pl.BlockSpec((TILE_R, 64), lambda i: (i, 0)),
            pl.BlockSpec((TILE_R, 128), lambda i: (i, 0)),
            pl.BlockSpec((8, 64), lambda i: (0, 0)),
        ],
        out_shape=[
            jax.ShapeDtypeStruct((N_PAD, 64), jnp.float32),
            jax.ShapeDtypeStruct((N_PAD, 128), jnp.float32),
            jax.ShapeDtypeStruct((8, 64), jnp.float32),
        ],
        scratch_shapes=[pltpu.VMEM((8, 64), jnp.float32)],
        compiler_params=_TC_PARAMS,
    )(x, degp, wc)


def _bn_tanh(parts, stats, g_ref, b_ref):
    h = jnp.concatenate(parts, axis=1)
    sm = jnp.concatenate([s[0:1, :] for s in stats], axis=1)
    sq = jnp.concatenate([s[1:2, :] for s in stats], axis=1)
    m = sm * (1.0 / N)
    v = sq * (1.0 / N) - m * m
    return jnp.tanh((h - m) * lax.rsqrt(v + EPS) * g_ref[...] + b_ref[...])


def _tc_layer(p0, p1, p2, s0, s1, s2, gp, bp, degp, wc):
    def body(p0_ref, p1_ref, p2_ref, s0_ref, s1_ref, s2_ref, g_ref, b_ref,
             deg_ref, w_ref, q0_ref, hs_ref, st_ref, sacc):
        i = pl.program_id(0)
        hn = _bn_tanh((p0_ref[...], p1_ref[...], p2_ref[...]),
                      (s0_ref, s1_ref, s2_ref), g_ref, b_ref)
        rows = i * TILE_R + lax.broadcasted_iota(jnp.int32, (TILE_R, 1), 0)
        hn = jnp.where(rows < N, hn, 0.0)
        p = jnp.dot(hn, w_ref[...], preferred_element_type=jnp.float32)
        q0 = p[:, 0:64]
        q0_ref[...] = q0
        dinv = _dinv_tile(deg_ref)
        hs_ref[...] = p[:, 64:192] * dinv
        _stats_update(sacc, q0, i)

        @pl.when(i == NT - 1)
        def _():
            st_ref[...] = sacc[...]

    part = pl.BlockSpec((TILE_R, 64), lambda i: (i, 0))
    st_in = pl.BlockSpec((8, 64), lambda i: (0, 0))
    vec = pl.BlockSpec((1, 192), lambda i: (0, 0))
    return pl.pallas_call(
        body,
        grid=(NT,),
        in_specs=[part, part, part, st_in, st_in, st_in, vec, vec,
                  pl.BlockSpec((2, TILE_R, 128), lambda i: (0, i, 0)),
                  pl.BlockSpec((192, 192), lambda i: (0, 0))],
        out_specs=[
            pl.BlockSpec((TILE_R, 64), lambda i: (i, 0)),
            pl.BlockSpec((TILE_R, 128), lambda i: (i, 0)),
            pl.BlockSpec((8, 64), lambda i: (0, 0)),
        ],
        out_shape=[
            jax.ShapeDtypeStruct((N_PAD, 64), jnp.float32),
            jax.ShapeDtypeStruct((N_PAD, 128), jnp.float32),
            jax.ShapeDtypeStruct((8, 64), jnp.float32),
        ],
        scratch_shapes=[pltpu.VMEM((8, 64), jnp.float32)],
        compiler_params=_TC_PARAMS,
    )(p0, p1, p2, s0, s1, s2, gp, bp, degp, wc)


def _tc_hop1(acc1, degp):
    def body(a_ref, deg_ref, q1_ref, hs_ref, st_ref, sacc):
        i = pl.program_id(0)
        dinv = _dinv_tile(deg_ref)
        r = (a_ref[0] + a_ref[1]) * dinv
        q1 = r[:, 0:64]
        q1_ref[...] = q1
        hs_ref[...] = jnp.concatenate(
            [r[:, 64:128] * dinv, jnp.zeros((TILE_R, 64), jnp.float32)], axis=1)
        _stats_update(sacc, q1, i)

        @pl.when(i == NT - 1)
        def _():
            st_ref[...] = sacc[...]

    return pl.pallas_call(
        body,
        grid=(NT,),
        in_specs=[
            pl.BlockSpec((2, TILE_R, 128), lambda i: (0, i, 0)),
            pl.BlockSpec((2, TILE_R, 128), lambda i: (0, i, 0)),
        ],
        out_specs=[
            pl.BlockSpec((TILE_R, 64), lambda i: (i, 0)),
            pl.BlockSpec((TILE_R, 128), lambda i: (i, 0)),
            pl.BlockSpec((8, 64), lambda i: (0, 0)),
        ],
        out_shape=[
            jax.ShapeDtypeStruct((N_PAD, 64), jnp.float32),
            jax.ShapeDtypeStruct((N_PAD, 128), jnp.float32),
            jax.ShapeDtypeStruct((8, 64), jnp.float32),
        ],
        scratch_shapes=[pltpu.VMEM((8, 64), jnp.float32)],
        compiler_params=_TC_PARAMS,
    )(acc1, degp)


def _tc_hop2(acc2, degp):
    def body(a_ref, deg_ref, q2_ref, st_ref, sacc):
        i = pl.program_id(0)
        dinv = _dinv_tile(deg_ref)
        q2 = (a_ref[0, :, 0:64] + a_ref[1, :, 0:64]) * dinv
        q2_ref[...] = q2
        _stats_update(sacc, q2, i)

        @pl.when(i == NT - 1)
        def _():
            st_ref[...] = sacc[...]

    return pl.pallas_call(
        body,
        grid=(NT,),
        in_specs=[
            pl.BlockSpec((2, TILE_R, 128), lambda i: (0, i, 0)),
            pl.BlockSpec((2, TILE_R, 128), lambda i: (0, i, 0)),
        ],
        out_specs=[
            pl.BlockSpec((TILE_R, 64), lambda i: (i, 0)),
            pl.BlockSpec((8, 64), lambda i: (0, 0)),
        ],
        out_shape=[
            jax.ShapeDtypeStruct((N_PAD, 64), jnp.float32),
            jax.ShapeDtypeStruct((8, 64), jnp.float32),
        ],
        scratch_shapes=[pltpu.VMEM((8, 64), jnp.float32)],
        compiler_params=_TC_PARAMS,
    )(acc2, degp)


def _tc_out(p0, p1, p2, s0, s1, s2, gp, bp, wout):
    def body(p0_ref, p1_ref, p2_ref, s0_ref, s1_ref, s2_ref, g_ref, b_ref,
             w_ref, y_ref):
        hn = _bn_tanh((p0_ref[...], p1_ref[...], p2_ref[...]),
                      (s0_ref, s1_ref, s2_ref), g_ref, b_ref)
        y_ref[...] = jnp.dot(hn, w_ref[...], preferred_element_type=jnp.float32)

    part = pl.BlockSpec((TILE_R, 64), lambda i: (i, 0))
    st_in = pl.BlockSpec((8, 64), lambda i: (0, 0))
    vec = pl.BlockSpec((1, 192), lambda i: (0, 0))
    return pl.pallas_call(
        body,
        grid=(NT,),
        in_specs=[part, part, part, st_in, st_in, st_in, vec, vec,
                  pl.BlockSpec((192, OUT), lambda i: (0, 0))],
        out_specs=pl.BlockSpec((TILE_R, OUT), lambda i: (i, 0)),
        out_shape=jax.ShapeDtypeStruct((N_PAD, OUT), jnp.float32),
        compiler_params=_TC_PARAMS,
    )(p0, p1, p2, s0, s1, s2, gp, bp, wout)


def _pack_cols(w0, w1, w2):
    z = jnp.zeros((w0.shape[0], 4), jnp.float32)
    return jnp.concatenate([w0, z, w1, z, w2, z], axis=1)


def _pack_rows(w):
    z = jnp.zeros((4, w.shape[1]), jnp.float32)
    return jnp.concatenate(
        [w[0:60], z, w[60:120], z, w[120:180], z], axis=0)


def _pack_vec(v):
    z = jnp.zeros((4,), jnp.float32)
    return jnp.concatenate(
        [v[0:60], z, v[60:120], z, v[120:180], z]).reshape(1, 192)


def kernel(x, edge_index, W1_0, W1_1, W1_2, g1, b1, W2_0, W2_1, W2_2, g2, b2,
           W3_0, W3_1, W3_2, g3, b3, W_out):
    x_pad = jnp.zeros((N_PAD, D_IN), jnp.float32).at[:N].set(x)
    pad = N + (jnp.arange(E_PAD - E, dtype=jnp.int32) % (N_PAD - N))
    srcr = jnp.concatenate([edge_index[0], pad]).reshape(NW, NCH, CH)
    dstr = jnp.concatenate([edge_index[1], pad]).reshape(NW, NCH, CH)

    wc1 = _pack_cols(W1_0, W1_1, W1_2)
    wc2 = _pack_rows(_pack_cols(W2_0, W2_1, W2_2))
    wc3 = _pack_rows(_pack_cols(W3_0, W3_1, W3_2))
    wop = _pack_rows(W_out)

    degp = _propagate(srcr, dstr)

    q0, hs, s0 = _tc_first(x_pad, degp, wc1)
    a1 = _propagate(srcr, dstr, hs)
    q1, hs2, s1 = _tc_hop1(a1, degp)
    a2 = _propagate(srcr, dstr, hs2)
    q2, s2 = _tc_hop2(a2, degp)

    for gcur, bcur, wc in ((g1, b1, wc2), (g2, b2, wc3)):
        q0, hs, s0n = _tc_layer(q0, q1, q2, s0, s1, s2,
                                _pack_vec(gcur), _pack_vec(bcur), degp, wc)
        a1 = _propagate(srcr, dstr, hs)
        q1, hs2, s1 = _tc_hop1(a1, degp)
        a2 = _propagate(srcr, dstr, hs2)
        q2, s2 = _tc_hop2(a2, degp)
        s0 = s0n

    y = _tc_out(q0, q1, q2, s0, s1, s2, _pack_vec(g3), _pack_vec(b3), wop)
    return y[:N]


# speedup vs baseline: 4.1363x; 1.0016x over previous
import functools

import jax
import jax.numpy as jnp
from jax import lax
from jax.experimental import pallas as pl
from jax.experimental.pallas import tpu as pltpu
from jax.experimental.pallas import tpu_sc as plsc

N = 10000
N_PAD = 10240
E = 320000
D_IN = 128
HID = 60
OUT = 64
EPS = 1e-5

NC = 2
NS = 16
NW = NC * NS
CH = 128
NCH = 80
IB = 16
NB = NCH // IB
SL = 2
E_PAD = NW * CH * NCH
STRIPE = N_PAD // NS

TILE_R = 1024
NT = N_PAD // TILE_R


def _sc_mesh():
    return plsc.VectorSubcoreMesh(core_axis_name="c", subcore_axis_name="s")


def _fill(rows_ref, value, width):
    vec = jnp.full((16,), value, jnp.float32)

    def body(i, _):
        for k in range(width // 16):
            rows_ref[i, pl.ds(k * 16, 16)] = vec
        return 0

    lax.fori_loop(0, CH, body, 0)


def _propagate(srcr, dstr, hs=None):
    W = 128
    gather = hs is not None
    ins = (hs, srcr, dstr) if gather else (srcr, dstr)
    if gather:
        scratch = [
            pltpu.VMEM((2, IB, CH), jnp.int32),
            pltpu.VMEM((NCH, CH), jnp.int32),
            pltpu.VMEM((CH, W), jnp.float32),
            pltpu.VMEM((CH, W), jnp.float32),
            pltpu.VMEM_SHARED((N_PAD, W), jnp.float32),
            pltpu.SemaphoreType.DMA,
            pltpu.SemaphoreType.DMA,
            pltpu.SemaphoreType.DMA,
            pltpu.SemaphoreType.DMA,
        ]
    else:
        scratch = [
            pltpu.VMEM((NCH, CH), jnp.int32),
            pltpu.VMEM((CH, W), jnp.float32),
            pltpu.VMEM_SHARED((N_PAD, W), jnp.float32),
            pltpu.SemaphoreType.DMA,
            pltpu.SemaphoreType.DMA,
        ]

    @functools.partial(
        pl.kernel,
        out_type=jax.ShapeDtypeStruct((NC, N_PAD, W), jnp.float32),
        mesh=_sc_mesh(),
        scratch_types=scratch,
    )
    def k(*refs):
        if gather:
            hs_ref, srcr_ref, dstr_ref, out_ref = refs[:4]
            sidxb, didx, rows0, rows1, acc, gsem0, gsem1, isem0, isem1 = refs[4:]
            rows = rows0
        else:
            srcr_ref, dstr_ref, out_ref = refs[:3]
            didx, rows, acc, ssem0, ssem1 = refs[3:]
        c = lax.axis_index("c")
        s = lax.axis_index("s")
        wid = c * NS + s

        if gather:
            isem = (isem0, isem1)

            def sidx_fetch(kb, slot):
                pltpu.async_copy(srcr_ref.at[wid, pl.ds(kb * IB, IB)],
                                 sidxb.at[slot], isem[slot])

            sidx_fetch(0, 0)
        pltpu.sync_copy(dstr_ref.at[wid], didx)
        _fill(rows, 0.0, W)
        for r in range(STRIPE // CH):
            pltpu.sync_copy(rows, acc.at[pl.ds(s * STRIPE + r * CH, CH)])
        if not gather:
            _fill(rows, 1.0, W)
        plsc.subcore_barrier()

        if gather:
            rbuf = (rows0, rows1)
            gsem = (gsem0, gsem1)

            def gath(slot, l, b):
                pltpu.async_copy(hs_ref.at[sidxb.at[slot, l]], rbuf[b],
                                 gsem[b])

            def wait_g(b):
                pltpu.make_async_copy(hs_ref.at[sidxb.at[0, 0]], rbuf[b],
                                      gsem[b]).wait()

            def scat(j, b):
                pltpu.sync_copy(rbuf[b], acc.at[didx.at[j]], add=True)

            for kb in range(NB):
                slot = kb % 2
                pltpu.make_async_copy(srcr_ref.at[wid, pl.ds(0, IB)],
                                      sidxb.at[slot], isem[slot]).wait()
                if kb + 1 < NB:
                    sidx_fetch(kb + 1, 1 - slot)
                base = kb * IB
                gath(slot, 0, 0)
                gath(slot, 1, 1)

                def pair(a, _):
                    wait_g(0)
                    scat(base + 2 * a, 0)
                    gath(slot, 2 * a + 2, 0)
                    wait_g(1)
                    scat(base + 2 * a + 1, 1)
                    gath(slot, 2 * a + 3, 1)
                    return 0

                lax.fori_loop(0, IB // 2 - 1, pair, 0)
                wait_g(0)
                scat(base + IB - 2, 0)
                wait_g(1)
                scat(base + IB - 1, 1)
        else:
            ssem = (ssem0, ssem1)

            def scat_a(j, b):
                pltpu.async_copy(rows, acc.at[didx.at[j]], ssem[b], add=True)

            def wait_s(b):
                pltpu.make_async_copy(rows, acc.at[didx.at[0]],
                                      ssem[b]).wait()

            scat_a(0, 0)
            scat_a(1, 1)

            def chunk(a, _):
                wait_s(0)
                scat_a(2 * a + 2, 0)
                wait_s(1)
                scat_a(2 * a + 3, 1)
                return 0

            lax.fori_loop(0, NCH // 2 - 1, chunk, 0)
            wait_s(0)
            wait_s(1)

        plsc.subcore_barrier()
        pltpu.sync_copy(acc.at[pl.ds(s * STRIPE, STRIPE)],
                        out_ref.at[c, pl.ds(s * STRIPE, STRIPE)])

    return k(*ins)


_TC_PARAMS = pltpu.CompilerParams(dimension_semantics=("arbitrary",))


def _dinv_tile(deg_ref):
    d = deg_ref[0, :, 0:1] + deg_ref[1, :, 0:1]
    return lax.rsqrt(jnp.maximum(d, 1.0))


def _stats_update(sacc, q, i):
    @pl.when(i == 0)
    def _():
        sacc[...] = jnp.zeros_like(sacc)

    sacc[0:1, :] += jnp.sum(q, axis=0, keepdims=True)
    sacc[1:2, :] += jnp.sum(q * q, axis=0, keepdims=True)


def _tc_proj1(x, wc):
    def body(x_ref, w_ref, p_ref, q0_ref, st_ref, sacc):
        i = pl.program_id(0)
        p = jnp.dot(x_ref[...], w_ref[...], preferred_element_type=jnp.float32)
        q0 = p[:, 0:64]
        p_ref[...] = p
        q0_ref[...] = q0
        _stats_update(sacc, q0, i)

        @pl.when(i == NT - 1)
        def _():
            st_ref[...] = sacc[...]

    return pl.pallas_call(
        body,
        grid=(NT,),
        in_specs=[
            pl.BlockSpec((TILE_R, D_IN), lambda i: (i, 0)),
            pl.BlockSpec((D_IN, 192), lambda i: (0, 0)),
        ],
        out_specs=[
            pl.BlockSpec((TILE_R, 192), lambda i: (i, 0)),
            pl.BlockSpec((TILE_R, 64), lambda i: (i, 0)),
            pl.BlockSpec((8, 64), lambda i: (0, 0)),
        ],
        out_shape=[
            jax.ShapeDtypeStruct((N_PAD, 192), jnp.float32),
            jax.ShapeDtypeStruct((N_PAD, 64), jnp.float32),
            jax.ShapeDtypeStruct((8, 64), jnp.float32),
        ],
        scratch_shapes=[pltpu.VMEM((8, 64), jnp.float32)],
        compiler_params=_TC_PARAMS,
    )(x, wc)


def _tc_scale1(p, degp):
    def body(p_ref, deg_ref, hs_ref):
        dinv = _dinv_tile(deg_ref)
        hs_ref[...] = p_ref[...] * dinv

    return pl.pallas_call(
        body,
        grid=(NT,),
        in_specs=[
            pl.BlockSpec((TILE_R, 128), lambda i: (i, 0)),
            pl.BlockSpec((2, TILE_R, 128), lambda i: (0, i, 0)),
        ],
        out_specs=pl.BlockSpec((TILE_R, 128), lambda i: (i, 0)),
        out_shape=jax.ShapeDtypeStruct((N_PAD, 128), jnp.float32),
        compiler_params=_TC_PARAMS,
    )(jax.lax.slice(p, (0, 64), (N_PAD, 192)), degp)


def _bn_tanh(parts, stats, g_ref, b_ref):
    h = jnp.concatenate(parts, axis=1)
    sm = jnp.concatenate([s[0:1, :] for s in stats], axis=1)
    sq = jnp.concatenate([s[1:2, :] for s in stats], axis=1)
    m = sm * (1.0 / N)
    v = sq * (1.0 / N) - m * m
    return jnp.tanh((h - m) * lax.rsqrt(v + EPS) * g_ref[...] + b_ref[...])


def _tc_layer(p0, p1, p2, s0, s1, s2, gp, bp, degp, wc):
    def body(p0_ref, p1_ref, p2_ref, s0_ref, s1_ref, s2_ref, g_ref, b_ref,
             deg_ref, w_ref, q0_ref, hs_ref, st_ref, sacc):
        i = pl.program_id(0)
        hn = _bn_tanh((p0_ref[...], p1_ref[...], p2_ref[...]),
                      (s0_ref, s1_ref, s2_ref), g_ref, b_ref)
        rows = i * TILE_R + lax.broadcasted_iota(jnp.int32, (TILE_R, 1), 0)
        hn = jnp.where(rows < N, hn, 0.0)
        p = jnp.dot(hn, w_ref[...], preferred_element_type=jnp.float32)
        q0 = p[:, 0:64]
        q0_ref[...] = q0
        dinv = _dinv_tile(deg_ref)
        hs_ref[...] = p[:, 64:192] * dinv
        _stats_update(sacc, q0, i)

        @pl.when(i == NT - 1)
        def _():
            st_ref[...] = sacc[...]

    part = pl.BlockSpec((TILE_R, 64), lambda i: (i, 0))
    st_in = pl.BlockSpec((8, 64), lambda i: (0, 0))
    vec = pl.BlockSpec((1, 192), lambda i: (0, 0))
    return pl.pallas_call(
        body,
        grid=(NT,),
        in_specs=[part, part, part, st_in, st_in, st_in, vec, vec,
                  pl.BlockSpec((2, TILE_R, 128), lambda i: (0, i, 0)),
                  pl.BlockSpec((192, 192), lambda i: (0, 0))],
        out_specs=[
            pl.BlockSpec((TILE_R, 64), lambda i: (i, 0)),
            pl.BlockSpec((TILE_R, 128), lambda i: (i, 0)),
            pl.BlockSpec((8, 64), lambda i: (0, 0)),
        ],
        out_shape=[
            jax.ShapeDtypeStruct((N_PAD, 64), jnp.float32),
            jax.ShapeDtypeStruct((N_PAD, 128), jnp.float32),
            jax.ShapeDtypeStruct((8, 64), jnp.float32),
        ],
        scratch_shapes=[pltpu.VMEM((8, 64), jnp.float32)],
        compiler_params=_TC_PARAMS,
    )(p0, p1, p2, s0, s1, s2, gp, bp, degp, wc)


def _tc_hop1(acc1, degp):
    def body(a_ref, deg_ref, q1_ref, hs_ref, st_ref, sacc):
        i = pl.program_id(0)
        dinv = _dinv_tile(deg_ref)
        r = (a_ref[0] + a_ref[1]) * dinv
        q1 = r[:, 0:64]
        q1_ref[...] = q1
        hs_ref[...] = jnp.concatenate(
            [r[:, 64:128] * dinv, jnp.zeros((TILE_R, 64), jnp.float32)], axis=1)
        _stats_update(sacc, q1, i)

        @pl.when(i == NT - 1)
        def _():
            st_ref[...] = sacc[...]

    return pl.pallas_call(
        body,
        grid=(NT,),
        in_specs=[
            pl.BlockSpec((2, TILE_R, 128), lambda i: (0, i, 0)),
            pl.BlockSpec((2, TILE_R, 128), lambda i: (0, i, 0)),
        ],
        out_specs=[
            pl.BlockSpec((TILE_R, 64), lambda i: (i, 0)),
            pl.BlockSpec((TILE_R, 128), lambda i: (i, 0)),
            pl.BlockSpec((8, 64), lambda i: (0, 0)),
        ],
        out_shape=[
            jax.ShapeDtypeStruct((N_PAD, 64), jnp.float32),
            jax.ShapeDtypeStruct((N_PAD, 128), jnp.float32),
            jax.ShapeDtypeStruct((8, 64), jnp.float32),
        ],
        scratch_shapes=[pltpu.VMEM((8, 64), jnp.float32)],
        compiler_params=_TC_PARAMS,
    )(acc1, degp)


def _tc_hop2(acc2, degp):
    def body(a_ref, deg_ref, q2_ref, st_ref, sacc):
        i = pl.program_id(0)
        dinv = _dinv_tile(deg_ref)
        q2 = (a_ref[0, :, 0:64] + a_ref[1, :, 0:64]) * dinv
        q2_ref[...] = q2
        _stats_update(sacc, q2, i)

        @pl.when(i == NT - 1)
        def _():
            st_ref[...] = sacc[...]

    return pl.pallas_call(
        body,
        grid=(NT,),
        in_specs=[
            pl.BlockSpec((2, TILE_R, 128), lambda i: (0, i, 0)),
            pl.BlockSpec((2, TILE_R, 128), lambda i: (0, i, 0)),
        ],
        out_specs=[
            pl.BlockSpec((TILE_R, 64), lambda i: (i, 0)),
            pl.BlockSpec((8, 64), lambda i: (0, 0)),
        ],
        out_shape=[
            jax.ShapeDtypeStruct((N_PAD, 64), jnp.float32),
            jax.ShapeDtypeStruct((8, 64), jnp.float32),
        ],
        scratch_shapes=[pltpu.VMEM((8, 64), jnp.float32)],
        compiler_params=_TC_PARAMS,
    )(acc2, degp)


def _tc_out(p0, p1, p2, s0, s1, s2, gp, bp, wout):
    def body(p0_ref, p1_ref, p2_ref, s0_ref, s1_ref, s2_ref, g_ref, b_ref,
             w_ref, y_ref):
        hn = _bn_tanh((p0_ref[...], p1_ref[...], p2_ref[...]),
                      (s0_ref, s1_ref, s2_ref), g_ref, b_ref)
        y_ref[...] = jnp.dot(hn, w_ref[...], preferred_element_type=jnp.float32)

    part = pl.BlockSpec((TILE_R, 64), lambda i: (i, 0))
    st_in = pl.BlockSpec((8, 64), lambda i: (0, 0))
    vec = pl.BlockSpec((1, 192), lambda i: (0, 0))
    return pl.pallas_call(
        body,
        grid=(NT,),
        in_specs=[part, part, part, st_in, st_in, st_in, vec, vec,
                  pl.BlockSpec((192, OUT), lambda i: (0, 0))],
        out_specs=pl.BlockSpec((TILE_R, OUT), lambda i: (i, 0)),
        out_shape=jax.ShapeDtypeStruct((N_PAD, OUT), jnp.float32),
        compiler_params=_TC_PARAMS,
    )(p0, p1, p2, s0, s1, s2, gp, bp, wout)


def _pack_cols(w0, w1, w2):
    z = jnp.zeros((w0.shape[0], 4), jnp.float32)
    return jnp.concatenate([w0, z, w1, z, w2, z], axis=1)


def _pack_rows(w):
    z = jnp.zeros((4, w.shape[1]), jnp.float32)
    return jnp.concatenate(
        [w[0:60], z, w[60:120], z, w[120:180], z], axis=0)


def _pack_vec(v):
    z = jnp.zeros((4,), jnp.float32)
    return jnp.concatenate(
        [v[0:60], z, v[60:120], z, v[120:180], z]).reshape(1, 192)


def kernel(x, edge_index, W1_0, W1_1, W1_2, g1, b1, W2_0, W2_1, W2_2, g2, b2,
           W3_0, W3_1, W3_2, g3, b3, W_out):
    x_pad = jnp.zeros((N_PAD, D_IN), jnp.float32).at[:N].set(x)
    pad = N + (jnp.arange(E_PAD - E, dtype=jnp.int32) % (N_PAD - N))
    srcr = jnp.concatenate([edge_index[0], pad]).reshape(NW, NCH, CH)
    dstr = jnp.concatenate([edge_index[1], pad]).reshape(NW, NCH, CH)

    wc1 = _pack_cols(W1_0, W1_1, W1_2)
    wc2 = _pack_rows(_pack_cols(W2_0, W2_1, W2_2))
    wc3 = _pack_rows(_pack_cols(W3_0, W3_1, W3_2))
    wop = _pack_rows(W_out)

    degp = _propagate(srcr, dstr)

    p1, q0, s0 = _tc_proj1(x_pad, wc1)
    hs = _tc_scale1(p1, degp)
    a1 = _propagate(srcr, dstr, hs)
    q1, hs2, s1 = _tc_hop1(a1, degp)
    a2 = _propagate(srcr, dstr, hs2)
    q2, s2 = _tc_hop2(a2, degp)

    for gcur, bcur, wc in ((g1, b1, wc2), (g2, b2, wc3)):
        q0, hs, s0n = _tc_layer(q0, q1, q2, s0, s1, s2,
                                _pack_vec(gcur), _pack_vec(bcur), degp, wc)
        a1 = _propagate(srcr, dstr, hs)
        q1, hs2, s1 = _tc_hop1(a1, degp)
        a2 = _propagate(srcr, dstr, hs2)
        q2, s2 = _tc_hop2(a2, degp)
        s0 = s0n

    y = _tc_out(q0, q1, q2, s0, s1, s2, _pack_vec(g3), _pack_vec(b3), wop)
    return y[:N]
